# baseline (speedup 1.0000x reference)
import jax
import jax.numpy as jnp
from jax import lax
from jax.experimental import pallas as pl
from jax.experimental.pallas import tpu as pltpu
from jax.experimental.pallas import tpu_sc as plsc

NUM_TYPES = 1000
CONT_DIM = 4
ID_EMBED_DIM = 8
OUT_DIM = CONT_DIM + ID_EMBED_DIM

NC, NS, L = 2, 16, 16
NW = NC * NS
LANES = 128

CH = 4


def _sc_body(cont_hbm, idx_hbm, tab_hbm, out_hbm, tab_v,
             idx_v0, cont_v0, emb_v0, idx_v1, cont_v1, emb_v1,
             ld0, co0, eo0, ld1, co1, eo1):
    rows = idx_hbm.shape[0]
    per_w = rows // NW
    groups_w = per_w // 8
    n_iter = groups_w // CH
    rows_ch = CH * 8

    wid = lax.axis_index("s") * NC + lax.axis_index("c")
    g0 = wid * groups_w

    bufs = ((idx_v0, cont_v0, emb_v0, ld0, co0, eo0),
            (idx_v1, cont_v1, emb_v1, ld1, co1, eo1))

    def start_load(k, b):
        kc = jnp.minimum(k, n_iter - 1)
        m0 = g0 + kc * CH
        idxv, contv, _, lds, _, _ = bufs[b]
        pltpu.async_copy(idx_hbm.at[pl.ds(m0 * 8, rows_ch), :], idxv, lds)
        for g in range(CH):
            m = m0 + g
            tt = m // LANES
            bb = lax.rem(m, LANES)
            pltpu.async_copy(cont_hbm.at[pl.ds(tt * 8, 8), bb, :, :],
                             contv.at[g], lds)

    def wait_load(b):
        idxv, contv, _, lds, _, _ = bufs[b]
        pltpu.make_async_copy(idx_hbm.at[pl.ds(0, rows_ch), :], idxv, lds).wait()
        for g in range(CH):
            pltpu.make_async_copy(cont_hbm.at[pl.ds(0, 8), 0, :, :],
                                  contv.at[g], lds).wait()

    def fire_co(k, b):
        _, contv, _, _, cos, _ = bufs[b]
        m0 = g0 + k * CH
        for g in range(CH):
            m = m0 + g
            for c in range(CONT_DIM):
                pltpu.async_copy(contv.at[g, :, c, :],
                                 out_hbm.at[c, pl.ds(m * 8, 8), :], cos)

    def wait_co(b):
        _, contv, _, _, cos, _ = bufs[b]
        for _i in range(CH * CONT_DIM):
            pltpu.make_async_copy(contv.at[0, :, 0, :],
                                  out_hbm.at[0, pl.ds(0, 8), :], cos).wait()

    def compute(b):
        idxv, _, embv, _, _, _ = bufs[b]

        def row_body(r, carry):
            for sub in range(LANES // L):
                ids8 = idxv[r, pl.ds(sub * L, L)] * ID_EMBED_DIM
                vals = [plsc.load_gather(tab_v, [ids8 + e])
                        for e in range(ID_EMBED_DIM)]
                for e in range(ID_EMBED_DIM):
                    embv[e, r, pl.ds(sub * L, L)] = vals[e]
            return carry

        lax.fori_loop(0, rows_ch, row_body, 0)

    def fire_eo(k, b):
        _, _, embv, _, _, eos = bufs[b]
        row0 = (g0 + k * CH) * 8
        for e in range(ID_EMBED_DIM):
            pltpu.async_copy(embv.at[e],
                             out_hbm.at[CONT_DIM + e, pl.ds(row0, rows_ch), :],
                             eos)

    def wait_eo(b):
        _, _, embv, _, _, eos = bufs[b]
        for e in range(ID_EMBED_DIM):
            pltpu.make_async_copy(embv.at[e],
                                  out_hbm.at[CONT_DIM, pl.ds(0, rows_ch), :],
                                  eos).wait()

    def phase(k, b, first):
        wait_load(b)
        fire_co(k, b)
        if not first:
            wait_eo(b)
        compute(b)
        fire_eo(k, b)
        wait_co(b)
        start_load(k + 2, b)

    pltpu.sync_copy(tab_hbm, tab_v)

    start_load(0, 0)
    start_load(1, 1)
    phase(0, 0, True)
    phase(1, 1, True)

    def loop_body(j, carry):
        phase(2 * j, 0, False)
        phase(2 * j + 1, 1, False)
        return carry

    lax.fori_loop(1, (n_iter - 1) // 2, loop_body, 0)
    phase(n_iter - 1, 0, False)

    wait_load(0)
    wait_load(1)
    wait_eo(0)
    wait_eo(1)


def kernel(tokens_cont, tokens_id, id_embedding):
    B, T, _ = tokens_cont.shape
    n_tok = B * T
    rows = n_tok // LANES
    bb_n = B // LANES

    cont_lin = tokens_cont.reshape(bb_n, LANES, T, CONT_DIM).transpose(2, 0, 3, 1)
    idx_lin = tokens_id.astype(jnp.int32).reshape(bb_n, LANES, T // 8, 8).transpose(
        2, 0, 3, 1).reshape(rows, LANES)
    tab_flat = id_embedding.reshape(NUM_TYPES * ID_EMBED_DIM)

    mesh = plsc.VectorSubcoreMesh(core_axis_name="c", subcore_axis_name="s")
    out = pl.kernel(
        _sc_body,
        out_type=jax.ShapeDtypeStruct((OUT_DIM, rows, LANES), jnp.float32),
        mesh=mesh,
        scratch_types=[
            pltpu.VMEM((NUM_TYPES * ID_EMBED_DIM,), jnp.float32),
            pltpu.VMEM((CH * 8, LANES), jnp.int32),
            pltpu.VMEM((CH, 8, CONT_DIM, LANES), jnp.float32),
            pltpu.VMEM((ID_EMBED_DIM, CH * 8, LANES), jnp.float32),
            pltpu.VMEM((CH * 8, LANES), jnp.int32),
            pltpu.VMEM((CH, 8, CONT_DIM, LANES), jnp.float32),
            pltpu.VMEM((ID_EMBED_DIM, CH * 8, LANES), jnp.float32),
            pltpu.SemaphoreType.DMA,
            pltpu.SemaphoreType.DMA,
            pltpu.SemaphoreType.DMA,
            pltpu.SemaphoreType.DMA,
            pltpu.SemaphoreType.DMA,
            pltpu.SemaphoreType.DMA,
        ],
        compiler_params=pltpu.CompilerParams(needs_layout_passes=False),
    )(cont_lin, idx_lin, tab_flat)

    return out.reshape(OUT_DIM, T // 8, bb_n, 8, LANES).transpose(
        2, 4, 1, 3, 0).reshape(B, T, OUT_DIM)


# speedup vs baseline: 1.0282x; 1.0282x over previous
import jax
import jax.numpy as jnp
from jax import lax
from jax.experimental import pallas as pl
from jax.experimental.pallas import tpu as pltpu
from jax.experimental.pallas import tpu_sc as plsc

NUM_TYPES = 1000
CONT_DIM = 4
ID_EMBED_DIM = 8
OUT_DIM = CONT_DIM + ID_EMBED_DIM

NC, NS, L = 2, 16, 16
NW = NC * NS
LANES = 128

CH = 4


def _sc_body(cont_hbm, idx_hbm, tab_hbm, out_hbm, tab_v,
             idx_v0, cont_v0, emb_v0, idx_v1, cont_v1, emb_v1,
             ld0, co0, eo0, ld1, co1, eo1):
    rows = idx_hbm.shape[0]
    per_w = rows // NW
    groups_w = per_w // 8
    n_iter = groups_w // CH
    rows_ch = CH * 8

    wid = lax.axis_index("s") * NC + lax.axis_index("c")
    g0 = wid * groups_w

    bufs = ((idx_v0, cont_v0, emb_v0, ld0, co0, eo0),
            (idx_v1, cont_v1, emb_v1, ld1, co1, eo1))

    def start_load(k, b):
        kc = jnp.minimum(k, n_iter - 1)
        m0 = g0 + kc * CH
        idxv, contv, _, lds, _, _ = bufs[b]
        pltpu.async_copy(idx_hbm.at[pl.ds(m0 * 8, rows_ch), :], idxv, lds)
        for g in range(CH):
            m = m0 + g
            tt = m // LANES
            bb = lax.rem(m, LANES)
            pltpu.async_copy(cont_hbm.at[pl.ds(tt * 8, 8), bb, :, :],
                             contv.at[g], lds)

    def wait_load(b):
        idxv, contv, _, lds, _, _ = bufs[b]
        pltpu.make_async_copy(idx_hbm.at[pl.ds(0, rows_ch), :], idxv, lds).wait()
        for g in range(CH):
            pltpu.make_async_copy(cont_hbm.at[pl.ds(0, 8), 0, :, :],
                                  contv.at[g], lds).wait()

    def fire_co(k, b):
        _, contv, _, _, cos, _ = bufs[b]
        m0 = g0 + k * CH
        for c in range(CONT_DIM):
            pltpu.async_copy(contv.at[:, :, c, :],
                             out_hbm.at[c, pl.ds(m0, CH), :, :], cos)

    def wait_co(b):
        _, contv, _, _, cos, _ = bufs[b]
        for _i in range(CONT_DIM):
            pltpu.make_async_copy(contv.at[:, :, 0, :],
                                  out_hbm.at[0, pl.ds(0, CH), :, :], cos).wait()

    def compute(b):
        idxv, _, embv, _, _, _ = bufs[b]

        def row_body(r, carry):
            for sub in range(LANES // L):
                ids8 = idxv[r, pl.ds(sub * L, L)] * ID_EMBED_DIM
                vals = [plsc.load_gather(tab_v, [ids8 + e])
                        for e in range(ID_EMBED_DIM)]
                for e in range(ID_EMBED_DIM):
                    embv[e, r // 8, lax.rem(r, 8), pl.ds(sub * L, L)] = vals[e]
            return carry

        lax.fori_loop(0, rows_ch, row_body, 0)

    def fire_eo(k, b):
        _, _, embv, _, _, eos = bufs[b]
        m0 = g0 + k * CH
        for e in range(ID_EMBED_DIM):
            pltpu.async_copy(embv.at[e],
                             out_hbm.at[CONT_DIM + e, pl.ds(m0, CH), :, :],
                             eos)

    def wait_eo(b):
        _, _, embv, _, _, eos = bufs[b]
        for e in range(ID_EMBED_DIM):
            pltpu.make_async_copy(embv.at[e],
                                  out_hbm.at[CONT_DIM, pl.ds(0, CH), :, :],
                                  eos).wait()

    def phase(k, b, first):
        wait_load(b)
        fire_co(k, b)
        if not first:
            wait_eo(b)
        compute(b)
        fire_eo(k, b)
        wait_co(b)
        start_load(k + 2, b)

    pltpu.sync_copy(tab_hbm, tab_v)

    start_load(0, 0)
    start_load(1, 1)
    phase(0, 0, True)
    phase(1, 1, True)

    def loop_body(j, carry):
        phase(2 * j, 0, False)
        phase(2 * j + 1, 1, False)
        return carry

    lax.fori_loop(1, (n_iter - 1) // 2, loop_body, 0)
    phase(n_iter - 1, 0, False)

    wait_load(0)
    wait_load(1)
    wait_eo(0)
    wait_eo(1)


def kernel(tokens_cont, tokens_id, id_embedding):
    B, T, _ = tokens_cont.shape
    n_tok = B * T
    rows = n_tok // LANES
    bb_n = B // LANES

    cont_lin = tokens_cont.reshape(bb_n, LANES, T, CONT_DIM).transpose(2, 0, 3, 1)
    idx_lin = tokens_id.astype(jnp.int32).reshape(bb_n, LANES, T // 8, 8).transpose(
        2, 0, 3, 1).reshape(rows, LANES)
    tab_flat = id_embedding.reshape(NUM_TYPES * ID_EMBED_DIM)

    mesh = plsc.VectorSubcoreMesh(core_axis_name="c", subcore_axis_name="s")
    out = pl.kernel(
        _sc_body,
        out_type=jax.ShapeDtypeStruct((OUT_DIM, rows // 8, 8, LANES), jnp.float32),
        mesh=mesh,
        scratch_types=[
            pltpu.VMEM((NUM_TYPES * ID_EMBED_DIM,), jnp.float32),
            pltpu.VMEM((CH * 8, LANES), jnp.int32),
            pltpu.VMEM((CH, 8, CONT_DIM, LANES), jnp.float32),
            pltpu.VMEM((ID_EMBED_DIM, CH, 8, LANES), jnp.float32),
            pltpu.VMEM((CH * 8, LANES), jnp.int32),
            pltpu.VMEM((CH, 8, CONT_DIM, LANES), jnp.float32),
            pltpu.VMEM((ID_EMBED_DIM, CH, 8, LANES), jnp.float32),
            pltpu.SemaphoreType.DMA,
            pltpu.SemaphoreType.DMA,
            pltpu.SemaphoreType.DMA,
            pltpu.SemaphoreType.DMA,
            pltpu.SemaphoreType.DMA,
            pltpu.SemaphoreType.DMA,
        ],
        compiler_params=pltpu.CompilerParams(needs_layout_passes=False),
    )(cont_lin, idx_lin, tab_flat)

    return out.reshape(OUT_DIM, T // 8, bb_n, 8, LANES).transpose(
        2, 4, 1, 3, 0).reshape(B, T, OUT_DIM)


# speedup vs baseline: 1.0387x; 1.0101x over previous
import jax
import jax.numpy as jnp
from jax import lax
from jax.experimental import pallas as pl
from jax.experimental.pallas import tpu as pltpu
from jax.experimental.pallas import tpu_sc as plsc

NUM_TYPES = 1000
CONT_DIM = 4
ID_EMBED_DIM = 8
OUT_DIM = CONT_DIM + ID_EMBED_DIM

NC, NS, L = 2, 16, 16
NW = NC * NS
LANES = 128

CH = 4


def _sc_body(cont_hbm, idx_hbm, tab_hbm, out_hbm, tab_v,
             idx_v0, cont_v0, emb_v0, idx_v1, cont_v1, emb_v1,
             ld0, co0, eo0, ld1, co1, eo1):
    rows = idx_hbm.shape[0]
    per_w = rows // NW
    groups_w = per_w // 8
    n_iter = groups_w // CH
    rows_ch = CH * 8

    wid = lax.axis_index("s") * NC + lax.axis_index("c")
    g0 = wid * groups_w

    bufs = ((idx_v0, cont_v0, emb_v0, ld0, co0, eo0),
            (idx_v1, cont_v1, emb_v1, ld1, co1, eo1))

    def start_load(k, b):
        kc = jnp.minimum(k, n_iter - 1)
        m0 = g0 + kc * CH
        idxv, contv, _, lds, _, _ = bufs[b]
        pltpu.async_copy(idx_hbm.at[pl.ds(m0 * 8, rows_ch), :], idxv, lds)
        for g in range(CH):
            m = m0 + g
            tt = m // LANES
            bb = lax.rem(m, LANES)
            pltpu.async_copy(cont_hbm.at[pl.ds(tt * 8, 8), bb, :, :],
                             contv.at[g], lds)

    def wait_load(b):
        idxv, contv, _, lds, _, _ = bufs[b]
        pltpu.make_async_copy(idx_hbm.at[pl.ds(0, rows_ch), :], idxv, lds).wait()
        for g in range(CH):
            pltpu.make_async_copy(cont_hbm.at[pl.ds(0, 8), 0, :, :],
                                  contv.at[g], lds).wait()

    def fire_co(k, b):
        _, contv, _, _, cos, _ = bufs[b]
        m0 = g0 + k * CH
        for c in range(CONT_DIM):
            pltpu.async_copy(contv.at[:, :, c, :],
                             out_hbm.at[c, pl.ds(m0, CH), :, :], cos)

    def wait_co(b):
        _, contv, _, _, cos, _ = bufs[b]
        for _i in range(CONT_DIM):
            pltpu.make_async_copy(contv.at[:, :, 0, :],
                                  out_hbm.at[0, pl.ds(0, CH), :, :], cos).wait()

    def compute(b):
        idxv, _, embv, _, _, _ = bufs[b]

        def row_body(r2, carry):
            for half in range(2):
                r = r2 * 2 + half
                g = r // 8
                s = lax.rem(r, 8)
                for sub in range(LANES // L):
                    ids8 = idxv[r, pl.ds(sub * L, L)] * ID_EMBED_DIM
                    vals = [plsc.load_gather(tab_v, [ids8 + e])
                            for e in range(ID_EMBED_DIM)]
                    for e in range(ID_EMBED_DIM):
                        embv[e, g, s, pl.ds(sub * L, L)] = vals[e]
            return carry

        lax.fori_loop(0, rows_ch // 2, row_body, 0)

    def fire_eo(k, b):
        _, _, embv, _, _, eos = bufs[b]
        m0 = g0 + k * CH
        for e in range(ID_EMBED_DIM):
            pltpu.async_copy(embv.at[e],
                             out_hbm.at[CONT_DIM + e, pl.ds(m0, CH), :, :],
                             eos)

    def wait_eo(b):
        _, _, embv, _, _, eos = bufs[b]
        for e in range(ID_EMBED_DIM):
            pltpu.make_async_copy(embv.at[e],
                                  out_hbm.at[CONT_DIM, pl.ds(0, CH), :, :],
                                  eos).wait()

    def phase(k, b, first):
        wait_load(b)
        fire_co(k, b)
        if not first:
            wait_eo(b)
        compute(b)
        fire_eo(k, b)
        wait_co(b)
        start_load(k + 2, b)

    pltpu.sync_copy(tab_hbm, tab_v)

    start_load(0, 0)
    start_load(1, 1)
    phase(0, 0, True)
    phase(1, 1, True)

    def loop_body(j, carry):
        phase(2 * j, 0, False)
        phase(2 * j + 1, 1, False)
        return carry

    lax.fori_loop(1, (n_iter - 1) // 2, loop_body, 0)
    phase(n_iter - 1, 0, False)

    wait_load(0)
    wait_load(1)
    wait_eo(0)
    wait_eo(1)


def kernel(tokens_cont, tokens_id, id_embedding):
    B, T, _ = tokens_cont.shape
    n_tok = B * T
    rows = n_tok // LANES
    bb_n = B // LANES

    cont_lin = tokens_cont.reshape(bb_n, LANES, T, CONT_DIM).transpose(2, 0, 3, 1)
    idx_lin = tokens_id.astype(jnp.int32).reshape(bb_n, LANES, T // 8, 8).transpose(
        2, 0, 3, 1).reshape(rows, LANES)
    tab_flat = id_embedding.reshape(NUM_TYPES * ID_EMBED_DIM)

    mesh = plsc.VectorSubcoreMesh(core_axis_name="c", subcore_axis_name="s")
    out = pl.kernel(
        _sc_body,
        out_type=jax.ShapeDtypeStruct((OUT_DIM, rows // 8, 8, LANES), jnp.float32),
        mesh=mesh,
        scratch_types=[
            pltpu.VMEM((NUM_TYPES * ID_EMBED_DIM,), jnp.float32),
            pltpu.VMEM((CH * 8, LANES), jnp.int32),
            pltpu.VMEM((CH, 8, CONT_DIM, LANES), jnp.float32),
            pltpu.VMEM((ID_EMBED_DIM, CH, 8, LANES), jnp.float32),
            pltpu.VMEM((CH * 8, LANES), jnp.int32),
            pltpu.VMEM((CH, 8, CONT_DIM, LANES), jnp.float32),
            pltpu.VMEM((ID_EMBED_DIM, CH, 8, LANES), jnp.float32),
            pltpu.SemaphoreType.DMA,
            pltpu.SemaphoreType.DMA,
            pltpu.SemaphoreType.DMA,
            pltpu.SemaphoreType.DMA,
            pltpu.SemaphoreType.DMA,
            pltpu.SemaphoreType.DMA,
        ],
        compiler_params=pltpu.CompilerParams(needs_layout_passes=False),
    )(cont_lin, idx_lin, tab_flat)

    return out.reshape(OUT_DIM, T // 8, bb_n, 8, LANES).transpose(
        2, 4, 1, 3, 0).reshape(B, T, OUT_DIM)


# speedup vs baseline: 1.1458x; 1.1031x over previous
import jax
import jax.numpy as jnp
from jax import lax
from jax.experimental import pallas as pl
from jax.experimental.pallas import tpu as pltpu
from jax.experimental.pallas import tpu_sc as plsc

NUM_TYPES = 1000
CONT_DIM = 4
ID_EMBED_DIM = 8
OUT_DIM = CONT_DIM + ID_EMBED_DIM

NC, NS, L = 2, 16, 16
NW = NC * NS
LANES = 128

CH = 4


def _sc_body(cont_hbm, idx_hbm, tab_hbm, out_hbm, tab_v,
             idx_v0, cont_v0, emb_v0, idx_v1, cont_v1, emb_v1,
             ld0, co0, eo0, ld1, co1, eo1):
    rows = idx_hbm.shape[0]
    per_w = rows // NW
    groups_w = per_w // 8
    n_iter = groups_w // CH
    rows_ch = CH * 8

    wid = lax.axis_index("s") * NC + lax.axis_index("c")
    g0 = wid * groups_w

    bufs = ((idx_v0, cont_v0, emb_v0, ld0, co0, eo0),
            (idx_v1, cont_v1, emb_v1, ld1, co1, eo1))

    def start_load(k, b):
        kc = jnp.minimum(k, n_iter - 1)
        m0 = g0 + kc * CH
        idxv, contv, _, lds, _, _ = bufs[b]
        pltpu.async_copy(idx_hbm.at[pl.ds(m0 * 8, rows_ch), :], idxv, lds)
        for g in range(CH):
            m = m0 + g
            tt = m // LANES
            bb = lax.rem(m, LANES)
            pltpu.async_copy(cont_hbm.at[pl.ds(tt * 8, 8), bb, :, :],
                             contv.at[g], lds)

    def wait_load(b):
        idxv, contv, _, lds, _, _ = bufs[b]
        pltpu.make_async_copy(idx_hbm.at[pl.ds(0, rows_ch), :], idxv, lds).wait()
        for g in range(CH):
            pltpu.make_async_copy(cont_hbm.at[pl.ds(0, 8), 0, :, :],
                                  contv.at[g], lds).wait()

    def fire_co(k, b):
        _, contv, _, _, cos, _ = bufs[b]
        m0 = g0 + k * CH
        for c in range(CONT_DIM):
            pltpu.async_copy(contv.at[:, :, c, :],
                             out_hbm.at[c, pl.ds(m0, CH), :, :], cos)

    def wait_co(b):
        _, contv, _, _, cos, _ = bufs[b]
        for _i in range(CONT_DIM):
            pltpu.make_async_copy(contv.at[:, :, 0, :],
                                  out_hbm.at[0, pl.ds(0, CH), :, :], cos).wait()

    def compute(b):
        idxv, _, embv, _, _, _ = bufs[b]

        n_sub = LANES // L

        def row_body(r, carry):
            g = r // 8
            s = lax.rem(r, 8)
            ids8 = [idxv[r, pl.ds(sub * L, L)] * ID_EMBED_DIM
                    for sub in range(n_sub)]

            def gath(sub):
                return [plsc.load_gather(tab_v, [ids8[sub] + e])
                        for e in range(ID_EMBED_DIM)]

            prev = gath(0)
            for sub in range(1, n_sub):
                cur = gath(sub)
                for e in range(ID_EMBED_DIM):
                    embv[e, g, s, pl.ds((sub - 1) * L, L)] = prev[e]
                prev = cur
            for e in range(ID_EMBED_DIM):
                embv[e, g, s, pl.ds((n_sub - 1) * L, L)] = prev[e]
            return carry

        lax.fori_loop(0, rows_ch, row_body, 0)

    def fire_eo(k, b):
        _, _, embv, _, _, eos = bufs[b]
        m0 = g0 + k * CH
        for e in range(ID_EMBED_DIM):
            pltpu.async_copy(embv.at[e],
                             out_hbm.at[CONT_DIM + e, pl.ds(m0, CH), :, :],
                             eos)

    def wait_eo(b):
        _, _, embv, _, _, eos = bufs[b]
        for e in range(ID_EMBED_DIM):
            pltpu.make_async_copy(embv.at[e],
                                  out_hbm.at[CONT_DIM, pl.ds(0, CH), :, :],
                                  eos).wait()

    def phase(k, b, first):
        wait_load(b)
        fire_co(k, b)
        if not first:
            wait_eo(b)
        compute(b)
        fire_eo(k, b)
        wait_co(b)
        start_load(k + 2, b)

    pltpu.sync_copy(tab_hbm, tab_v)

    start_load(0, 0)
    start_load(1, 1)
    phase(0, 0, True)
    phase(1, 1, True)

    def loop_body(j, carry):
        phase(2 * j, 0, False)
        phase(2 * j + 1, 1, False)
        return carry

    lax.fori_loop(1, (n_iter - 1) // 2, loop_body, 0)
    phase(n_iter - 1, 0, False)

    wait_load(0)
    wait_load(1)
    wait_eo(0)
    wait_eo(1)


def kernel(tokens_cont, tokens_id, id_embedding):
    B, T, _ = tokens_cont.shape
    n_tok = B * T
    rows = n_tok // LANES
    bb_n = B // LANES

    cont_lin = tokens_cont.reshape(bb_n, LANES, T, CONT_DIM).transpose(2, 0, 3, 1)
    idx_lin = tokens_id.astype(jnp.int32).reshape(bb_n, LANES, T // 8, 8).transpose(
        2, 0, 3, 1).reshape(rows, LANES)
    tab_flat = id_embedding.reshape(NUM_TYPES * ID_EMBED_DIM)

    mesh = plsc.VectorSubcoreMesh(core_axis_name="c", subcore_axis_name="s")
    out = pl.kernel(
        _sc_body,
        out_type=jax.ShapeDtypeStruct((OUT_DIM, rows // 8, 8, LANES), jnp.float32),
        mesh=mesh,
        scratch_types=[
            pltpu.VMEM((NUM_TYPES * ID_EMBED_DIM,), jnp.float32),
            pltpu.VMEM((CH * 8, LANES), jnp.int32),
            pltpu.VMEM((CH, 8, CONT_DIM, LANES), jnp.float32),
            pltpu.VMEM((ID_EMBED_DIM, CH, 8, LANES), jnp.float32),
            pltpu.VMEM((CH * 8, LANES), jnp.int32),
            pltpu.VMEM((CH, 8, CONT_DIM, LANES), jnp.float32),
            pltpu.VMEM((ID_EMBED_DIM, CH, 8, LANES), jnp.float32),
            pltpu.SemaphoreType.DMA,
            pltpu.SemaphoreType.DMA,
            pltpu.SemaphoreType.DMA,
            pltpu.SemaphoreType.DMA,
            pltpu.SemaphoreType.DMA,
            pltpu.SemaphoreType.DMA,
        ],
        compiler_params=pltpu.CompilerParams(needs_layout_passes=False),
    )(cont_lin, idx_lin, tab_flat)

    return out.reshape(OUT_DIM, T // 8, bb_n, 8, LANES).transpose(
        2, 4, 1, 3, 0).reshape(B, T, OUT_DIM)
